# baseline (speedup 1.0000x reference)
import jax
import jax.numpy as jnp
from jax import lax
from jax.experimental import pallas as pl
from jax.experimental.pallas import tpu as pltpu
from jax.experimental.pallas import tpu_sc as plsc

STATIC = 32
DYNAMIC = 32

BIG_ROWS = 40
NBIG = 25
REM_START = NBIG * BIG_ROWS
REM_ROWS = 26
JOBS = 2 * NBIG


def _sc_body(dyn_ref, grid_ref, out_ref, buf0, buf1, bufr,
             isem, osem, rsem_i, rsem_o):
    c = lax.axis_index("c")
    s = lax.axis_index("s")
    wid = s * 2 + c

    bufs = (buf0, buf1)
    isems = lambda t: isem.at[t & 1]
    osems = lambda t: osem.at[t & 1]

    def plane_refs(p):
        src = grid_ref if p == 0 else dyn_ref
        dst_row = wid if p == 0 else STATIC + wid
        return src, dst_row

    def in_copy(t):
        p, ci = divmod(t, NBIG)
        src, _ = plane_refs(p)
        return pltpu.async_copy(
            src.at[pl.ds(wid, 1), pl.ds(ci * BIG_ROWS, BIG_ROWS)],
            bufs[t & 1], isems(t))

    def out_copy(t):
        p, ci = divmod(t, NBIG)
        _, dst_row = plane_refs(p)
        return pltpu.async_copy(
            bufs[t & 1],
            out_ref.at[pl.ds(dst_row, 1), pl.ds(ci * BIG_ROWS, BIG_ROWS)],
            osems(t))

    def rem_in(p):
        src, _ = plane_refs(p)
        return pltpu.async_copy(
            src.at[pl.ds(wid, 1), pl.ds(REM_START, REM_ROWS)], bufr,
            rsem_i.at[p & 1])

    def rem_out(p):
        _, dst_row = plane_refs(p)
        return pltpu.async_copy(
            bufr, out_ref.at[pl.ds(dst_row, 1), pl.ds(REM_START, REM_ROWS)],
            rsem_o.at[p & 1])

    for t in range(JOBS):
        p, ci = divmod(t, NBIG)
        src, dst_row = plane_refs(p)
        b = bufs[t & 1]
        pltpu.sync_copy(
            src.at[pl.ds(wid, 1), pl.ds(ci * BIG_ROWS, BIG_ROWS)], b)
        pltpu.sync_copy(
            b, out_ref.at[pl.ds(dst_row, 1), pl.ds(ci * BIG_ROWS, BIG_ROWS)])
    for p in range(2):
        src, dst_row = plane_refs(p)
        pltpu.sync_copy(src.at[pl.ds(wid, 1), pl.ds(REM_START, REM_ROWS)], bufr)
        pltpu.sync_copy(
            bufr, out_ref.at[pl.ds(dst_row, 1), pl.ds(REM_START, REM_ROWS)])


def kernel(new_dynamic_state, grid):
    enc, depth, width = grid.shape
    mesh = plsc.VectorSubcoreMesh(core_axis_name="c", subcore_axis_name="s")
    run = pl.kernel(
        _sc_body,
        out_type=jax.ShapeDtypeStruct((enc, depth, width), grid.dtype),
        mesh=mesh,
        scratch_types=[
            pltpu.VMEM((1, BIG_ROWS, width), grid.dtype),
            pltpu.VMEM((1, BIG_ROWS, width), grid.dtype),
            pltpu.VMEM((1, REM_ROWS, width), grid.dtype),
            pltpu.SemaphoreType.DMA((2,)),
            pltpu.SemaphoreType.DMA((2,)),
            pltpu.SemaphoreType.DMA((2,)),
            pltpu.SemaphoreType.DMA((2,)),
        ],
    )
    return run(new_dynamic_state, grid)


# speedup vs baseline: 1.0007x; 1.0007x over previous
import jax
import jax.numpy as jnp
from jax import lax
from jax.experimental import pallas as pl
from jax.experimental.pallas import tpu as pltpu
from jax.experimental.pallas import tpu_sc as plsc

STATIC = 32
DYNAMIC = 32

BIG_ROWS = 40
NBIG = 25
REM_START = NBIG * BIG_ROWS
REM_ROWS = 26
JOBS = 2 * NBIG


def _sc_body(dyn_ref, grid_ref, out_ref, buf0, buf1, bufr):
    c = lax.axis_index("c")
    s = lax.axis_index("s")
    wid = s * 2 + c

    bufs = (buf0, buf1)

    def plane_refs(p):
        src = grid_ref if p == 0 else dyn_ref
        dst_row = wid if p == 0 else STATIC + wid
        return src, dst_row

    for t in range(JOBS):
        p, ci = divmod(t, NBIG)
        src, dst_row = plane_refs(p)
        b = bufs[t & 1]
        pltpu.sync_copy(
            src.at[pl.ds(wid, 1), pl.ds(ci * BIG_ROWS, BIG_ROWS)], b)
        pltpu.sync_copy(
            b, out_ref.at[pl.ds(dst_row, 1), pl.ds(ci * BIG_ROWS, BIG_ROWS)])
    for p in range(2):
        src, dst_row = plane_refs(p)
        pltpu.sync_copy(src.at[pl.ds(wid, 1), pl.ds(REM_START, REM_ROWS)], bufr)
        pltpu.sync_copy(
            bufr, out_ref.at[pl.ds(dst_row, 1), pl.ds(REM_START, REM_ROWS)])


def kernel(new_dynamic_state, grid):
    enc, depth, width = grid.shape
    mesh = plsc.VectorSubcoreMesh(core_axis_name="c", subcore_axis_name="s")
    run = pl.kernel(
        _sc_body,
        out_type=jax.ShapeDtypeStruct((enc, depth, width), grid.dtype),
        mesh=mesh,
        scratch_types=[
            pltpu.VMEM((1, BIG_ROWS, width), grid.dtype),
            pltpu.VMEM((1, BIG_ROWS, width), grid.dtype),
            pltpu.VMEM((1, REM_ROWS, width), grid.dtype),
        ],
    )
    return run(new_dynamic_state, grid)
